# baseline (speedup 1.0000x reference)
import jax
import jax.numpy as jnp
from jax import lax
from jax.experimental import pallas as pl
from jax.experimental.pallas import tpu as pltpu
from jax.experimental.pallas import tpu_sc as plsc

N = 10000
E = 320000
D = 128
NUM_REL = 8
NUM_BASES = 2

NSUB = 16
NCORE = 2
CHUNK = 128
NBUF = 2
KGRP = 8
CPS = 160
NGRP = CPS // KGRP
EPS = CPS * CHUNK
EPAD = NSUB * EPS
ROWS_PER_SUB = 632
N_ACC = NSUB * ROWS_PER_SUB
FLUSH = (128, 128, 128, 128, 120)
TBL = NUM_BASES * NUM_REL * N


def _scale_body(w_ref, x_ref, o_ref):
    o_ref[:] = w_ref[pl.program_id(0)] * x_ref[:]


def _build_table(w_flat, x):
    return pl.pallas_call(
        _scale_body,
        grid=(NUM_BASES * NUM_REL,),
        in_specs=[
            pl.BlockSpec(memory_space=pltpu.SMEM),
            pl.BlockSpec((N, D), lambda i: (0, 0)),
        ],
        out_specs=pl.BlockSpec((N, D), lambda i: (i, 0)),
        out_shape=jax.ShapeDtypeStruct((TBL, D), jnp.float32),
    )(w_flat, x)


def _sc_body(table, comb, zeros, out, *refs):
    idx_v = list(refs[0:2])
    rows_v = list(refs[2:2 + NBUF])
    acc = refs[2 + NBUF]
    gsem = list(refs[2 + NBUF + 1:2 + NBUF + 1 + NBUF])

    c = lax.axis_index("c")
    s = lax.axis_index("s")
    base = s * ROWS_PER_SUB

    def load_block(ib, g):
        pltpu.sync_copy(comb.at[c, s, g], idx_v[ib])

    def start_gather(b, ib, k):
        pltpu.async_copy(table.at[idx_v[ib].at[2 * k]], rows_v[b], gsem[b])

    def wait_gather(b, ib, k):
        pltpu.make_async_copy(
            table.at[idx_v[ib].at[2 * k]], rows_v[b], gsem[b]
        ).wait()

    def scatter(b, ib, k):
        pltpu.sync_copy(rows_v[b], acc.at[idx_v[ib].at[2 * k + 1]], add=True)

    pltpu.sync_copy(zeros, rows_v[0])
    r = base
    for n in FLUSH:
        pltpu.sync_copy(rows_v[0].at[pl.ds(0, n)], acc.at[pl.ds(r, n)])
        r += n
    plsc.subcore_barrier()

    load_block(0, 0)
    start_gather(0, 0, 0)

    def body(i, carry):
        g0 = 2 * i
        for half in range(2):
            ib, ob = half, 1 - half
            for k in range(KGRP):
                b = k % 2
                wait_gather(b, ib, k)
                if k < KGRP - 1:
                    start_gather(1 - b, ib, k + 1)
                    if k == 0 and half == 0:
                        load_block(1, g0 + 1)
                    if k == 0 and half == 1:
                        @pl.when(g0 + 2 < NGRP)
                        def _():
                            load_block(0, g0 + 2)
                else:
                    if half == 0:
                        start_gather(1 - b, ob, 0)
                    else:
                        @pl.when(g0 + 2 < NGRP)
                        def _():
                            start_gather(1 - b, ob, 0)
                scatter(b, ib, k)
        return carry

    lax.fori_loop(0, CPS // (2 * KGRP), body, 0)
    plsc.subcore_barrier()

    r = base
    for n in FLUSH:
        pltpu.sync_copy(acc.at[pl.ds(r, n)], rows_v[0].at[pl.ds(0, n)])
        pltpu.sync_copy(rows_v[0].at[pl.ds(0, n)], out.at[c, pl.ds(r, n)])
        r += n


def _sc_aggregate(table, comb, zeros):
    mesh = plsc.VectorSubcoreMesh(core_axis_name="c", subcore_axis_name="s")
    fn = pl.kernel(
        _sc_body,
        mesh=mesh,
        out_type=jax.ShapeDtypeStruct((NCORE, N_ACC, D), jnp.float32),
        scratch_types=(
            [pltpu.VMEM((2 * KGRP, CHUNK), jnp.int32) for _ in range(2)]
            + [pltpu.VMEM((CHUNK, D), jnp.float32) for _ in range(NBUF)]
            + [pltpu.VMEM_SHARED((N_ACC, D), jnp.float32)]
            + [pltpu.SemaphoreType.DMA for _ in range(NBUF)]
        ),
    )
    return fn(table, comb, zeros)


def _update_body(a0, a1, x_ref, wrel, wroot, bias, o_ref):
    wr = wroot[0] + wroot[1]
    o_ref[:] = (
        jnp.dot(a0[:], wrel[0], preferred_element_type=jnp.float32)
        + jnp.dot(a1[:], wrel[1], preferred_element_type=jnp.float32)
        + jnp.dot(x_ref[:], wr, preferred_element_type=jnp.float32)
        + (bias[0] + bias[1])[None, :]
    )


def _dense_update(a0, a1, x, W_rel, W_root, bias):
    BR = 400
    return pl.pallas_call(
        _update_body,
        grid=(N // BR,),
        in_specs=[
            pl.BlockSpec((BR, D), lambda i: (i, 0)),
            pl.BlockSpec((BR, D), lambda i: (i, 0)),
            pl.BlockSpec((BR, D), lambda i: (i, 0)),
            pl.BlockSpec((NUM_BASES, D, D), lambda i: (0, 0, 0)),
            pl.BlockSpec((NUM_BASES, D, D), lambda i: (0, 0, 0)),
            pl.BlockSpec((NUM_BASES, D), lambda i: (0, 0)),
        ],
        out_specs=pl.BlockSpec((BR, D), lambda i: (i, 0)),
        out_shape=jax.ShapeDtypeStruct((N, D), jnp.float32),
    )(a0, a1, x, W_rel, W_root, bias)


def kernel(edge_type, x, edge_index, edge_w, W_rel, W_root, bias):
    src = edge_index[0].astype(jnp.int32)
    dst = edge_index[1].astype(jnp.int32)
    et = edge_type.astype(jnp.int32)

    gidx = et * N + src
    gidx2 = jnp.stack([gidx, gidx + NUM_REL * N])
    pad = EPAD - E
    gidx_p = jnp.pad(gidx2, ((0, 0), (0, pad))).reshape(NCORE, NSUB, CPS, 1, CHUNK)
    dst_p = jnp.pad(dst, (0, pad), constant_values=N).reshape(NSUB, CPS, 1, CHUNK)
    dst_p = jnp.broadcast_to(dst_p, (NCORE, NSUB, CPS, 1, CHUNK))
    comb = jnp.concatenate([gidx_p, dst_p], axis=3)
    comb = comb.reshape(NCORE, NSUB, NGRP, 2 * KGRP, CHUNK)

    w_flat = edge_w.reshape(NUM_BASES * NUM_REL)
    table = _build_table(w_flat, x)
    zeros = jnp.zeros((CHUNK, D), jnp.float32)

    aggr = _sc_aggregate(table, comb, zeros)
    a0 = aggr[0, :N]
    a1 = aggr[1, :N]
    return _dense_update(a0, a1, x, W_rel, W_root, bias)


# speedup vs baseline: 1.4922x; 1.4922x over previous
import jax
import jax.numpy as jnp
from jax import lax
from jax.experimental import pallas as pl
from jax.experimental.pallas import tpu as pltpu
from jax.experimental.pallas import tpu_sc as plsc

N = 10000
E = 320000
D = 128
NUM_REL = 8
NUM_BASES = 2

NSUB = 16
NCORE = 2
CHUNK = 128
NBUF = 2
CPS = 158
EPS = CPS * CHUNK
EPAD = NSUB * EPS
ROWS_PER_SUB = 632
N_ACC = NSUB * ROWS_PER_SUB
FLUSH = (128, 128, 128, 128, 120)
TBL = NUM_BASES * NUM_REL * N


def _scale_body(w_ref, x_ref, o_ref):
    o_ref[:] = w_ref[pl.program_id(0)] * x_ref[:]


def _build_table(w_flat, x):
    return pl.pallas_call(
        _scale_body,
        grid=(NUM_BASES * NUM_REL,),
        in_specs=[
            pl.BlockSpec(memory_space=pltpu.SMEM),
            pl.BlockSpec((N, D), lambda i: (0, 0)),
        ],
        out_specs=pl.BlockSpec((N, D), lambda i: (i, 0)),
        out_shape=jax.ShapeDtypeStruct((TBL, D), jnp.float32),
    )(w_flat, x)


def _sc_body(table, comb, zeros, out, *refs):
    idx_v = list(refs[0:NBUF])
    rows_v = list(refs[NBUF:2 * NBUF])
    acc = refs[2 * NBUF]
    gsem = list(refs[2 * NBUF + 1:2 * NBUF + 1 + NBUF])
    isem = list(refs[2 * NBUF + 1 + NBUF:2 * NBUF + 1 + 2 * NBUF])

    c = lax.axis_index("c")
    s = lax.axis_index("s")
    base = s * ROWS_PER_SUB

    def load_idx(b, j):
        pltpu.sync_copy(comb.at[c, s, j], idx_v[b])

    def start_idx(b, j):
        pltpu.async_copy(comb.at[c, s, j], idx_v[b], isem[b])

    def wait_idx(b, j):
        pltpu.make_async_copy(comb.at[c, s, j], idx_v[b], isem[b]).wait()

    def start_gather(b):
        pltpu.async_copy(table.at[idx_v[b].at[0]], rows_v[b], gsem[b])

    def wait_gather(b):
        pltpu.make_async_copy(table.at[idx_v[b].at[0]], rows_v[b], gsem[b]).wait()

    def scatter(b):
        pltpu.sync_copy(rows_v[b], acc.at[idx_v[b].at[1]], add=True)

    pltpu.sync_copy(zeros, rows_v[0])
    r = base
    for n in FLUSH:
        pltpu.sync_copy(rows_v[0].at[pl.ds(0, n)], acc.at[pl.ds(r, n)])
        r += n
    plsc.subcore_barrier()

    load_idx(0, 0)
    start_gather(0)
    start_idx(1, 1)

    def pair(j0, carry):
        for b in range(NBUF):
            j = j0 + b
            nb = (b + 1) % NBUF
            wait_gather(b)

            @pl.when(j + 1 < CPS)
            def _():
                wait_idx(nb, j + 1)
                start_gather(nb)

            scatter(b)

            @pl.when(j + 2 < CPS)
            def _():
                start_idx(b, j + 2)

        return carry

    lax.fori_loop(0, CPS // NBUF, lambda q, cr: pair(q * NBUF, cr), 0)
    plsc.subcore_barrier()

    r = base
    for n in FLUSH:
        pltpu.sync_copy(acc.at[pl.ds(r, n)], rows_v[0].at[pl.ds(0, n)])
        pltpu.sync_copy(rows_v[0].at[pl.ds(0, n)], out.at[c, pl.ds(r, n)])
        r += n


def _sc_aggregate(table, comb, zeros):
    mesh = plsc.VectorSubcoreMesh(core_axis_name="c", subcore_axis_name="s")
    fn = pl.kernel(
        _sc_body,
        mesh=mesh,
        out_type=jax.ShapeDtypeStruct((NCORE, N_ACC, D), jnp.float32),
        scratch_types=(
            [pltpu.VMEM((2, CHUNK), jnp.int32) for _ in range(NBUF)]
            + [pltpu.VMEM((CHUNK, D), jnp.float32) for _ in range(NBUF)]
            + [pltpu.VMEM_SHARED((N_ACC, D), jnp.float32)]
            + [pltpu.SemaphoreType.DMA for _ in range(2 * NBUF)]
        ),
    )
    return fn(table, comb, zeros)


def _update_body(a0, a1, x_ref, wrel, wroot, bias, o_ref):
    wr = wroot[0] + wroot[1]
    o_ref[:] = (
        jnp.dot(a0[:], wrel[0], preferred_element_type=jnp.float32)
        + jnp.dot(a1[:], wrel[1], preferred_element_type=jnp.float32)
        + jnp.dot(x_ref[:], wr, preferred_element_type=jnp.float32)
        + (bias[0] + bias[1])[None, :]
    )


def _dense_update(a0, a1, x, W_rel, W_root, bias):
    BR = 400
    return pl.pallas_call(
        _update_body,
        grid=(N // BR,),
        in_specs=[
            pl.BlockSpec((BR, D), lambda i: (i, 0)),
            pl.BlockSpec((BR, D), lambda i: (i, 0)),
            pl.BlockSpec((BR, D), lambda i: (i, 0)),
            pl.BlockSpec((NUM_BASES, D, D), lambda i: (0, 0, 0)),
            pl.BlockSpec((NUM_BASES, D, D), lambda i: (0, 0, 0)),
            pl.BlockSpec((NUM_BASES, D), lambda i: (0, 0)),
        ],
        out_specs=pl.BlockSpec((BR, D), lambda i: (i, 0)),
        out_shape=jax.ShapeDtypeStruct((N, D), jnp.float32),
    )(a0, a1, x, W_rel, W_root, bias)


def kernel(edge_type, x, edge_index, edge_w, W_rel, W_root, bias):
    src = edge_index[0].astype(jnp.int32)
    dst = edge_index[1].astype(jnp.int32)
    et = edge_type.astype(jnp.int32)

    gidx = et * N + src
    gidx2 = jnp.stack([gidx, gidx + NUM_REL * N])
    pad = EPAD - E
    gidx_p = jnp.pad(gidx2, ((0, 0), (0, pad))).reshape(NCORE, NSUB, CPS, 1, CHUNK)
    dst_p = jnp.pad(dst, (0, pad), constant_values=N).reshape(NSUB, CPS, 1, CHUNK)
    dst_p = jnp.broadcast_to(dst_p, (NCORE, NSUB, CPS, 1, CHUNK))
    comb = jnp.concatenate([gidx_p, dst_p], axis=3)

    w_flat = edge_w.reshape(NUM_BASES * NUM_REL)
    table = _build_table(w_flat, x)
    zeros = jnp.zeros((CHUNK, D), jnp.float32)

    aggr = _sc_aggregate(table, comb, zeros)
    a0 = aggr[0, :N]
    a1 = aggr[1, :N]
    return _dense_update(a0, a1, x, W_rel, W_root, bias)
